# Initial kernel scaffold; baseline (speedup 1.0000x reference)
import jax
import jax.numpy as jnp
from jax.experimental import pallas as pl


def kernel(node_feat, node_ids, edge_ids, num_nodes_per_edge, num_edges_per_node, num_neighbors, W0, W1):
    raise NotImplementedError("write your pallas kernel here")


# speedup vs baseline: 2.3389x; 2.3389x over previous
import dataclasses
import functools

import jax
import jax.numpy as jnp
from jax import lax
from jax.experimental import pallas as pl
from jax.experimental.pallas import tpu as pltpu
from jax.experimental.pallas import tpu_sc as plsc

N_NODES = 10000
N_EDGES = 5000
NNZ = 640000
DIM = 128

NUM_CORES = 2
NUM_SUBCORES = 16
NW = NUM_CORES * NUM_SUBCORES
CPW = DIM // NW
LANES = 16

CHUNK = 8000
NCHUNK = NNZ // CHUNK

_MESH = plsc.VectorSubcoreMesh(
    core_axis_name="c", subcore_axis_name="s",
    num_cores=NUM_CORES, num_subcores=NUM_SUBCORES)

_SC_PARAMS = pltpu.CompilerParams()
if "needs_layout_passes" in pltpu.CompilerParams.__dataclass_fields__:
    _SC_PARAMS = dataclasses.replace(_SC_PARAMS, needs_layout_passes=False)


def _sc_aggregate(xt, nid, eid):
    @functools.partial(
        pl.kernel,
        out_type=jax.ShapeDtypeStruct((NW, N_NODES * CPW), jnp.float32),
        mesh=_MESH,
        compiler_params=_SC_PARAMS,
        scratch_types=[
            pltpu.VMEM((N_NODES * CPW,), jnp.float32),
            pltpu.VMEM((N_EDGES * CPW,), jnp.float32),
            pltpu.VMEM((N_NODES * CPW,), jnp.float32),
            pltpu.VMEM((CHUNK,), jnp.int32),
            pltpu.VMEM((CHUNK,), jnp.int32),
        ],
    )
    def agg_kernel(xt_hbm, nid_hbm, eid_hbm, out_hbm, xl, eacc, nacc, nbuf, ebuf):
        wid = lax.axis_index("s") * NUM_CORES + lax.axis_index("c")

        pltpu.sync_copy(xt_hbm.at[wid], xl)

        zero = jnp.zeros((LANES,), jnp.float32)

        @pl.loop(0, N_EDGES * CPW, step=LANES)
        def _(i):
            eacc[pl.ds(i, LANES)] = zero

        @pl.loop(0, N_NODES * CPW, step=LANES)
        def _(i):
            nacc[pl.ds(i, LANES)] = zero

        @pl.loop(0, NCHUNK)
        def _(ci):
            pltpu.sync_copy(nid_hbm.at[pl.ds(ci * CHUNK, CHUNK)], nbuf)
            pltpu.sync_copy(eid_hbm.at[pl.ds(ci * CHUNK, CHUNK)], ebuf)

            @pl.loop(0, CHUNK, step=LANES)
            def _(i):
                bn = nbuf[pl.ds(i, LANES)] * CPW
                be = ebuf[pl.ds(i, LANES)] * CPW
                for col in range(CPW):
                    v = plsc.load_gather(xl, [bn + col])
                    plsc.addupdate_scatter(eacc, [be + col], v)

        @pl.loop(0, NCHUNK)
        def _(ci):
            pltpu.sync_copy(nid_hbm.at[pl.ds(ci * CHUNK, CHUNK)], nbuf)
            pltpu.sync_copy(eid_hbm.at[pl.ds(ci * CHUNK, CHUNK)], ebuf)

            @pl.loop(0, CHUNK, step=LANES)
            def _(i):
                bn = nbuf[pl.ds(i, LANES)] * CPW
                be = ebuf[pl.ds(i, LANES)] * CPW
                for col in range(CPW):
                    v = plsc.load_gather(eacc, [be + col])
                    plsc.addupdate_scatter(nacc, [bn + col], v)

        pltpu.sync_copy(nacc, out_hbm.at[wid])

    return agg_kernel(xt, nid, eid)


def _mm_body(x_ref, agg_ref, nnb_ref, nepn_ref, wt_ref, wb_ref, o_ref):
    s = (1.0 / nnb_ref[...]) * (1.0 / nepn_ref[...])
    a = agg_ref[...] * s
    h = jnp.dot(x_ref[...], wt_ref[...], preferred_element_type=jnp.float32)
    h = h + jnp.dot(a, wb_ref[...], preferred_element_type=jnp.float32)
    o_ref[...] = jnp.where(h >= 0, h, h * 0.01)


_ROWS_BLK = 1000


def _tc_layer(x, agg, nnb, nepn, W):
    wt = W[:DIM]
    wb = W[DIM:]
    grid = (N_NODES // _ROWS_BLK,)
    return pl.pallas_call(
        _mm_body,
        grid=grid,
        in_specs=[
            pl.BlockSpec((_ROWS_BLK, DIM), lambda i: (i, 0)),
            pl.BlockSpec((_ROWS_BLK, DIM), lambda i: (i, 0)),
            pl.BlockSpec((_ROWS_BLK, 1), lambda i: (i, 0)),
            pl.BlockSpec((_ROWS_BLK, 1), lambda i: (i, 0)),
            pl.BlockSpec((DIM, DIM), lambda i: (0, 0)),
            pl.BlockSpec((DIM, DIM), lambda i: (0, 0)),
        ],
        out_specs=pl.BlockSpec((_ROWS_BLK, DIM), lambda i: (i, 0)),
        out_shape=jax.ShapeDtypeStruct((N_NODES, DIM), jnp.float32),
    )(x, agg, nnb, nepn, wt, wb)


def kernel(node_feat, node_ids, edge_ids, num_nodes_per_edge,
           num_edges_per_node, num_neighbors, W0, W1):
    del num_nodes_per_edge
    nid = node_ids.astype(jnp.int32)
    eid = edge_ids.astype(jnp.int32)
    nnb = num_neighbors.reshape(N_NODES, 1)
    nepn = num_edges_per_node.reshape(N_NODES, 1)

    x = node_feat
    for W in (W0, W1):
        xt = x.reshape(N_NODES, NW, CPW).transpose(1, 0, 2).reshape(NW, N_NODES * CPW)
        aggt = _sc_aggregate(xt, nid, eid)
        agg = aggt.reshape(NW, N_NODES, CPW).transpose(1, 0, 2).reshape(N_NODES, DIM)
        x = _tc_layer(x, agg, nnb, nepn, W)
    return x


# speedup vs baseline: 2.4063x; 1.0288x over previous
import dataclasses
import functools

import jax
import jax.numpy as jnp
from jax import lax
from jax.experimental import pallas as pl
from jax.experimental.pallas import tpu as pltpu
from jax.experimental.pallas import tpu_sc as plsc

N_NODES = 10000
N_EDGES = 5000
NNZ = 640000
DIM = 128

NUM_CORES = 2
NUM_SUBCORES = 16
NW = NUM_CORES * NUM_SUBCORES
CPW = DIM // NW
LANES = 16

CHUNK = 8000
NCHUNK = NNZ // CHUNK

_MESH = plsc.VectorSubcoreMesh(
    core_axis_name="c", subcore_axis_name="s",
    num_cores=NUM_CORES, num_subcores=NUM_SUBCORES)

_SC_PARAMS = pltpu.CompilerParams()
if "needs_layout_passes" in pltpu.CompilerParams.__dataclass_fields__:
    _SC_PARAMS = dataclasses.replace(_SC_PARAMS, needs_layout_passes=False)


def _sc_aggregate(xt, nid, eid):
    @functools.partial(
        pl.kernel,
        out_type=jax.ShapeDtypeStruct((NW, N_NODES * CPW), jnp.float32),
        mesh=_MESH,
        compiler_params=_SC_PARAMS,
        scratch_types=[
            pltpu.VMEM((N_NODES * CPW,), jnp.float32),
            pltpu.VMEM((N_EDGES * CPW,), jnp.float32),
            pltpu.VMEM((N_NODES * CPW,), jnp.float32),
            pltpu.VMEM((CHUNK,), jnp.int32),
            pltpu.VMEM((CHUNK,), jnp.int32),
        ],
    )
    def agg_kernel(xt_hbm, nid_hbm, eid_hbm, out_hbm, xl, eacc, nacc, nbuf, ebuf):
        wid = lax.axis_index("s") * NUM_CORES + lax.axis_index("c")

        pltpu.sync_copy(xt_hbm.at[wid], xl)

        zero = jnp.zeros((LANES,), jnp.float32)

        @pl.loop(0, N_EDGES * CPW, step=LANES)
        def _(i):
            eacc[pl.ds(i, LANES)] = zero

        @pl.loop(0, N_NODES * CPW, step=LANES)
        def _(i):
            nacc[pl.ds(i, LANES)] = zero

        @pl.loop(0, NCHUNK)
        def _(ci):
            pltpu.sync_copy(nid_hbm.at[pl.ds(ci * CHUNK, CHUNK)], nbuf)
            pltpu.sync_copy(eid_hbm.at[pl.ds(ci * CHUNK, CHUNK)], ebuf)

            @pl.loop(0, CHUNK, step=LANES, unroll=8)
            def _(i):
                bn = nbuf[pl.ds(i, LANES)] * CPW
                be = ebuf[pl.ds(i, LANES)] * CPW
                for col in range(CPW):
                    v = plsc.load_gather(xl, [bn + col])
                    plsc.addupdate_scatter(eacc, [be + col], v)

        @pl.loop(0, NCHUNK)
        def _(ci):
            pltpu.sync_copy(nid_hbm.at[pl.ds(ci * CHUNK, CHUNK)], nbuf)
            pltpu.sync_copy(eid_hbm.at[pl.ds(ci * CHUNK, CHUNK)], ebuf)

            @pl.loop(0, CHUNK, step=LANES, unroll=8)
            def _(i):
                bn = nbuf[pl.ds(i, LANES)] * CPW
                be = ebuf[pl.ds(i, LANES)] * CPW
                for col in range(CPW):
                    v = plsc.load_gather(eacc, [be + col])
                    plsc.addupdate_scatter(nacc, [bn + col], v)

        pltpu.sync_copy(nacc, out_hbm.at[wid])

    return agg_kernel(xt, nid, eid)


def _mm_body(x_ref, agg_ref, nnb_ref, nepn_ref, wt_ref, wb_ref, o_ref):
    s = (1.0 / nnb_ref[...]) * (1.0 / nepn_ref[...])
    a = agg_ref[...] * s
    h = jnp.dot(x_ref[...], wt_ref[...], preferred_element_type=jnp.float32)
    h = h + jnp.dot(a, wb_ref[...], preferred_element_type=jnp.float32)
    o_ref[...] = jnp.where(h >= 0, h, h * 0.01)


_ROWS_BLK = 1000


def _tc_layer(x, agg, nnb, nepn, W):
    wt = W[:DIM]
    wb = W[DIM:]
    grid = (N_NODES // _ROWS_BLK,)
    return pl.pallas_call(
        _mm_body,
        grid=grid,
        in_specs=[
            pl.BlockSpec((_ROWS_BLK, DIM), lambda i: (i, 0)),
            pl.BlockSpec((_ROWS_BLK, DIM), lambda i: (i, 0)),
            pl.BlockSpec((_ROWS_BLK, 1), lambda i: (i, 0)),
            pl.BlockSpec((_ROWS_BLK, 1), lambda i: (i, 0)),
            pl.BlockSpec((DIM, DIM), lambda i: (0, 0)),
            pl.BlockSpec((DIM, DIM), lambda i: (0, 0)),
        ],
        out_specs=pl.BlockSpec((_ROWS_BLK, DIM), lambda i: (i, 0)),
        out_shape=jax.ShapeDtypeStruct((N_NODES, DIM), jnp.float32),
    )(x, agg, nnb, nepn, wt, wb)


def kernel(node_feat, node_ids, edge_ids, num_nodes_per_edge,
           num_edges_per_node, num_neighbors, W0, W1):
    del num_nodes_per_edge
    nid = node_ids.astype(jnp.int32)
    eid = edge_ids.astype(jnp.int32)
    nnb = num_neighbors.reshape(N_NODES, 1)
    nepn = num_edges_per_node.reshape(N_NODES, 1)

    x = node_feat
    for W in (W0, W1):
        xt = x.reshape(N_NODES, NW, CPW).transpose(1, 0, 2).reshape(NW, N_NODES * CPW)
        aggt = _sc_aggregate(xt, nid, eid)
        agg = aggt.reshape(NW, N_NODES, CPW).transpose(1, 0, 2).reshape(N_NODES, DIM)
        x = _tc_layer(x, agg, nnb, nepn, W)
    return x


# speedup vs baseline: 4.8303x; 2.0073x over previous
import dataclasses
import functools

import jax
import jax.numpy as jnp
from jax import lax
from jax.experimental import pallas as pl
from jax.experimental.pallas import tpu as pltpu
from jax.experimental.pallas import tpu_sc as plsc

N_NODES = 10000
N_EDGES = 5000
NNZ = 640000
DIM = 128

NUM_CORES = 2
NUM_SUBCORES = 16
NW = NUM_CORES * NUM_SUBCORES
CPW = DIM // NW
LANES = 16

CHUNK = 8000
NCHUNK = NNZ // CHUNK

_MESH = plsc.VectorSubcoreMesh(
    core_axis_name="c", subcore_axis_name="s",
    num_cores=NUM_CORES, num_subcores=NUM_SUBCORES)

_SC_PARAMS = pltpu.CompilerParams()
if "needs_layout_passes" in pltpu.CompilerParams.__dataclass_fields__:
    _SC_PARAMS = dataclasses.replace(_SC_PARAMS, needs_layout_passes=False)


def _sc_aggregate(xt, nid, eid):
    @functools.partial(
        pl.kernel,
        out_type=jax.ShapeDtypeStruct((NW, N_NODES * CPW), jnp.float32),
        mesh=_MESH,
        compiler_params=_SC_PARAMS,
        scratch_types=[
            pltpu.VMEM((N_NODES * CPW,), jnp.float32),
            pltpu.VMEM((N_EDGES * CPW,), jnp.float32),
            pltpu.VMEM((N_NODES * CPW,), jnp.float32),
            pltpu.VMEM((CHUNK,), jnp.int32),
            pltpu.VMEM((CHUNK,), jnp.int32),
        ],
    )
    def agg_kernel(xt_hbm, nid_hbm, eid_hbm, out_hbm, xl, eacc, nacc, nbuf, ebuf):
        wid = lax.axis_index("s") * NUM_CORES + lax.axis_index("c")

        pltpu.sync_copy(xt_hbm.at[wid], xl)

        zero = jnp.zeros((LANES,), jnp.float32)

        @pl.loop(0, N_EDGES * CPW, step=LANES)
        def _(i):
            eacc[pl.ds(i, LANES)] = zero

        @pl.loop(0, N_NODES * CPW, step=LANES)
        def _(i):
            nacc[pl.ds(i, LANES)] = zero

        @pl.loop(0, NCHUNK)
        def _(ci):
            pltpu.sync_copy(nid_hbm.at[pl.ds(ci * CHUNK, CHUNK)], nbuf)
            pltpu.sync_copy(eid_hbm.at[pl.ds(ci * CHUNK, CHUNK)], ebuf)

            @plsc.parallel_loop(0, CHUNK, step=LANES, unroll=4)
            def _(i):
                bn = nbuf[pl.ds(i, LANES)] * CPW
                be = ebuf[pl.ds(i, LANES)] * CPW
                vs = [plsc.load_gather(xl, [bn + col]) for col in range(CPW)]
                for col in range(CPW):
                    plsc.addupdate_scatter(eacc, [be + col], vs[col])

        @pl.loop(0, NCHUNK)
        def _(ci):
            pltpu.sync_copy(nid_hbm.at[pl.ds(ci * CHUNK, CHUNK)], nbuf)
            pltpu.sync_copy(eid_hbm.at[pl.ds(ci * CHUNK, CHUNK)], ebuf)

            @plsc.parallel_loop(0, CHUNK, step=LANES, unroll=4)
            def _(i):
                bn = nbuf[pl.ds(i, LANES)] * CPW
                be = ebuf[pl.ds(i, LANES)] * CPW
                vs = [plsc.load_gather(eacc, [be + col]) for col in range(CPW)]
                for col in range(CPW):
                    plsc.addupdate_scatter(nacc, [bn + col], vs[col])

        pltpu.sync_copy(nacc, out_hbm.at[wid])

    return agg_kernel(xt, nid, eid)


def _mm_body(x_ref, agg_ref, nnb_ref, nepn_ref, wt_ref, wb_ref, o_ref):
    s = (1.0 / nnb_ref[...]) * (1.0 / nepn_ref[...])
    a = agg_ref[...] * s
    h = jnp.dot(x_ref[...], wt_ref[...], preferred_element_type=jnp.float32)
    h = h + jnp.dot(a, wb_ref[...], preferred_element_type=jnp.float32)
    o_ref[...] = jnp.where(h >= 0, h, h * 0.01)


_ROWS_BLK = 1000


def _tc_layer(x, agg, nnb, nepn, W):
    wt = W[:DIM]
    wb = W[DIM:]
    grid = (N_NODES // _ROWS_BLK,)
    return pl.pallas_call(
        _mm_body,
        grid=grid,
        in_specs=[
            pl.BlockSpec((_ROWS_BLK, DIM), lambda i: (i, 0)),
            pl.BlockSpec((_ROWS_BLK, DIM), lambda i: (i, 0)),
            pl.BlockSpec((_ROWS_BLK, 1), lambda i: (i, 0)),
            pl.BlockSpec((_ROWS_BLK, 1), lambda i: (i, 0)),
            pl.BlockSpec((DIM, DIM), lambda i: (0, 0)),
            pl.BlockSpec((DIM, DIM), lambda i: (0, 0)),
        ],
        out_specs=pl.BlockSpec((_ROWS_BLK, DIM), lambda i: (i, 0)),
        out_shape=jax.ShapeDtypeStruct((N_NODES, DIM), jnp.float32),
    )(x, agg, nnb, nepn, wt, wb)


def kernel(node_feat, node_ids, edge_ids, num_nodes_per_edge,
           num_edges_per_node, num_neighbors, W0, W1):
    del num_nodes_per_edge
    nid = node_ids.astype(jnp.int32)
    eid = edge_ids.astype(jnp.int32)
    nnb = num_neighbors.reshape(N_NODES, 1)
    nepn = num_edges_per_node.reshape(N_NODES, 1)

    x = node_feat
    for W in (W0, W1):
        xt = x.reshape(N_NODES, NW, CPW).transpose(1, 0, 2).reshape(NW, N_NODES * CPW)
        aggt = _sc_aggregate(xt, nid, eid)
        agg = aggt.reshape(NW, N_NODES, CPW).transpose(1, 0, 2).reshape(N_NODES, DIM)
        x = _tc_layer(x, agg, nnb, nepn, W)
    return x


# speedup vs baseline: 6.6935x; 1.3857x over previous
import dataclasses
import functools

import jax
import jax.numpy as jnp
from jax import lax
from jax.experimental import pallas as pl
from jax.experimental.pallas import tpu as pltpu
from jax.experimental.pallas import tpu_sc as plsc

N_NODES = 10000
N_EDGES = 5000
NNZ = 640000
DIM = 128

NUM_CORES = 2
NUM_SUBCORES = 16
NW = NUM_CORES * NUM_SUBCORES
CPW = DIM // NW
LANES = 16

CHUNK = 8000
NCHUNK = NNZ // CHUNK

_MESH = plsc.VectorSubcoreMesh(
    core_axis_name="c", subcore_axis_name="s",
    num_cores=NUM_CORES, num_subcores=NUM_SUBCORES)

_SC_PARAMS = pltpu.CompilerParams()
if "needs_layout_passes" in pltpu.CompilerParams.__dataclass_fields__:
    _SC_PARAMS = dataclasses.replace(_SC_PARAMS, needs_layout_passes=False)


def _sc_aggregate(xt, nid, eid):
    @functools.partial(
        pl.kernel,
        out_type=jax.ShapeDtypeStruct((NW, N_NODES * CPW), jnp.float32),
        mesh=_MESH,
        compiler_params=_SC_PARAMS,
        scratch_types=[
            pltpu.VMEM((N_NODES * CPW,), jnp.float32),
            pltpu.VMEM((N_EDGES * CPW,), jnp.float32),
            pltpu.VMEM((N_NODES * CPW,), jnp.float32),
            pltpu.VMEM((CHUNK,), jnp.int32),
            pltpu.VMEM((CHUNK,), jnp.int32),
        ],
    )
    def agg_kernel(xt_hbm, nid_hbm, eid_hbm, out_hbm, xl, eacc, nacc, nbuf, ebuf):
        wid = lax.axis_index("s") * NUM_CORES + lax.axis_index("c")

        pltpu.sync_copy(xt_hbm.at[wid], xl)

        zero = jnp.zeros((LANES,), jnp.float32)

        @pl.loop(0, N_EDGES * CPW, step=LANES)
        def _(i):
            eacc[pl.ds(i, LANES)] = zero

        @pl.loop(0, N_NODES * CPW, step=LANES)
        def _(i):
            nacc[pl.ds(i, LANES)] = zero

        @pl.loop(0, NCHUNK)
        def _(ci):
            pltpu.sync_copy(nid_hbm.at[pl.ds(ci * CHUNK, CHUNK)], nbuf)
            pltpu.sync_copy(eid_hbm.at[pl.ds(ci * CHUNK, CHUNK)], ebuf)

            @plsc.parallel_loop(0, CHUNK, step=LANES, unroll=4)
            def _(i):
                bn = nbuf[pl.ds(i, LANES)]
                be = ebuf[pl.ds(i, LANES)]
                vs = [plsc.load_gather(xl, [bn + col * N_NODES])
                      for col in range(CPW)]
                for col in range(CPW):
                    plsc.addupdate_scatter(eacc, [be + col * N_EDGES], vs[col])

        @pl.loop(0, NCHUNK)
        def _(ci):
            pltpu.sync_copy(nid_hbm.at[pl.ds(ci * CHUNK, CHUNK)], nbuf)
            pltpu.sync_copy(eid_hbm.at[pl.ds(ci * CHUNK, CHUNK)], ebuf)

            @plsc.parallel_loop(0, CHUNK, step=LANES, unroll=4)
            def _(i):
                bn = nbuf[pl.ds(i, LANES)]
                be = ebuf[pl.ds(i, LANES)]
                vs = [plsc.load_gather(eacc, [be + col * N_EDGES])
                      for col in range(CPW)]
                for col in range(CPW):
                    plsc.addupdate_scatter(nacc, [bn + col * N_NODES], vs[col])

        pltpu.sync_copy(nacc, out_hbm.at[wid])

    return agg_kernel(xt, nid, eid)


def _mm_body(x_ref, agg_ref, nnb_ref, nepn_ref, wt_ref, wb_ref, o_ref):
    s = (1.0 / nnb_ref[...]) * (1.0 / nepn_ref[...])
    a = agg_ref[...] * s
    h = jnp.dot(x_ref[...], wt_ref[...], preferred_element_type=jnp.float32)
    h = h + jnp.dot(a, wb_ref[...], preferred_element_type=jnp.float32)
    o_ref[...] = jnp.where(h >= 0, h, h * 0.01)


_ROWS_BLK = 1000


def _tc_layer(x, agg, nnb, nepn, W):
    wt = W[:DIM]
    wb = W[DIM:]
    grid = (N_NODES // _ROWS_BLK,)
    return pl.pallas_call(
        _mm_body,
        grid=grid,
        in_specs=[
            pl.BlockSpec((_ROWS_BLK, DIM), lambda i: (i, 0)),
            pl.BlockSpec((_ROWS_BLK, DIM), lambda i: (i, 0)),
            pl.BlockSpec((_ROWS_BLK, 1), lambda i: (i, 0)),
            pl.BlockSpec((_ROWS_BLK, 1), lambda i: (i, 0)),
            pl.BlockSpec((DIM, DIM), lambda i: (0, 0)),
            pl.BlockSpec((DIM, DIM), lambda i: (0, 0)),
        ],
        out_specs=pl.BlockSpec((_ROWS_BLK, DIM), lambda i: (i, 0)),
        out_shape=jax.ShapeDtypeStruct((N_NODES, DIM), jnp.float32),
    )(x, agg, nnb, nepn, wt, wb)


def kernel(node_feat, node_ids, edge_ids, num_nodes_per_edge,
           num_edges_per_node, num_neighbors, W0, W1):
    del num_nodes_per_edge
    nid = node_ids.astype(jnp.int32)
    eid = edge_ids.astype(jnp.int32)
    nnb = num_neighbors.reshape(N_NODES, 1)
    nepn = num_edges_per_node.reshape(N_NODES, 1)

    x = node_feat
    for W in (W0, W1):
        xt = x.reshape(N_NODES, NW, CPW).transpose(1, 2, 0).reshape(NW, N_NODES * CPW)
        aggt = _sc_aggregate(xt, nid, eid)
        agg = aggt.reshape(NW, CPW, N_NODES).transpose(2, 0, 1).reshape(N_NODES, DIM)
        x = _tc_layer(x, agg, nnb, nepn, W)
    return x


# speedup vs baseline: 9.9895x; 1.4924x over previous
import dataclasses
import functools

import jax
import jax.numpy as jnp
from jax import lax
from jax.experimental import pallas as pl
from jax.experimental.pallas import tpu as pltpu
from jax.experimental.pallas import tpu_sc as plsc

N_NODES = 10000
N_EDGES = 5000
NNZ = 640000
DIM = 128

NUM_CORES = 2
NUM_SUBCORES = 16
NW = NUM_CORES * NUM_SUBCORES
CPW = DIM // NW
LANES = 16

CHUNK = 4000
NCHUNK = NNZ // CHUNK

_MESH = plsc.VectorSubcoreMesh(
    core_axis_name="c", subcore_axis_name="s",
    num_cores=NUM_CORES, num_subcores=NUM_SUBCORES)

_SC_PARAMS = pltpu.CompilerParams()
if "needs_layout_passes" in pltpu.CompilerParams.__dataclass_fields__:
    _SC_PARAMS = dataclasses.replace(_SC_PARAMS, needs_layout_passes=False)


def _sc_aggregate(xt, nid, eid):
    @functools.partial(
        pl.kernel,
        out_type=jax.ShapeDtypeStruct((NW, N_NODES * CPW), jnp.float32),
        mesh=_MESH,
        compiler_params=_SC_PARAMS,
        scratch_types=[
            pltpu.VMEM((N_NODES * CPW,), jnp.float32),
            pltpu.VMEM((N_EDGES * CPW,), jnp.float32),
            pltpu.VMEM((N_NODES * CPW,), jnp.float32),
        ],
    )
    def agg_kernel(xt_hbm, nid_hbm, eid_hbm, out_hbm, xl, eacc, nacc):
        wid = lax.axis_index("s") * NUM_CORES + lax.axis_index("c")

        pltpu.sync_copy(xt_hbm.at[wid], xl)

        zero = jnp.zeros((LANES,), jnp.float32)

        @pl.loop(0, N_EDGES * CPW, step=LANES)
        def _(i):
            eacc[pl.ds(i, LANES)] = zero

        @pl.loop(0, N_NODES * CPW, step=LANES)
        def _(i):
            nacc[pl.ds(i, LANES)] = zero

        def pass_a_body(nbuf, ebuf):
            @plsc.parallel_loop(0, CHUNK, step=LANES, unroll=4)
            def _(i):
                bn = nbuf[pl.ds(i, LANES)]
                be = ebuf[pl.ds(i, LANES)]
                vs = [plsc.load_gather(xl, [bn + col * N_NODES])
                      for col in range(CPW)]
                for col in range(CPW):
                    plsc.addupdate_scatter(eacc, [be + col * N_EDGES], vs[col])

        pltpu.emit_pipeline(
            pass_a_body,
            grid=(NCHUNK,),
            in_specs=[pl.BlockSpec((CHUNK,), lambda i: (i,)),
                      pl.BlockSpec((CHUNK,), lambda i: (i,))],
            out_specs=[],
        )(nid_hbm, eid_hbm)

        def pass_b_body(nbuf, ebuf):
            @plsc.parallel_loop(0, CHUNK, step=LANES, unroll=4)
            def _(i):
                bn = nbuf[pl.ds(i, LANES)]
                be = ebuf[pl.ds(i, LANES)]
                vs = [plsc.load_gather(eacc, [be + col * N_EDGES])
                      for col in range(CPW)]
                for col in range(CPW):
                    plsc.addupdate_scatter(nacc, [bn + col * N_NODES], vs[col])

        pltpu.emit_pipeline(
            pass_b_body,
            grid=(NCHUNK,),
            in_specs=[pl.BlockSpec((CHUNK,), lambda i: (i,)),
                      pl.BlockSpec((CHUNK,), lambda i: (i,))],
            out_specs=[],
        )(nid_hbm, eid_hbm)

        pltpu.sync_copy(nacc, out_hbm.at[wid])

    return agg_kernel(xt, nid, eid)


def _mm_body(x_ref, agg_ref, nnb_ref, nepn_ref, wt_ref, wb_ref, o_ref):
    s = (1.0 / nnb_ref[...]) * (1.0 / nepn_ref[...])
    a = agg_ref[...] * s
    h = jnp.dot(x_ref[...], wt_ref[...], preferred_element_type=jnp.float32)
    h = h + jnp.dot(a, wb_ref[...], preferred_element_type=jnp.float32)
    o_ref[...] = jnp.where(h >= 0, h, h * 0.01)


_ROWS_BLK = 1000


def _tc_layer(x, agg, nnb, nepn, W):
    wt = W[:DIM]
    wb = W[DIM:]
    grid = (N_NODES // _ROWS_BLK,)
    return pl.pallas_call(
        _mm_body,
        grid=grid,
        in_specs=[
            pl.BlockSpec((_ROWS_BLK, DIM), lambda i: (i, 0)),
            pl.BlockSpec((_ROWS_BLK, DIM), lambda i: (i, 0)),
            pl.BlockSpec((_ROWS_BLK, 1), lambda i: (i, 0)),
            pl.BlockSpec((_ROWS_BLK, 1), lambda i: (i, 0)),
            pl.BlockSpec((DIM, DIM), lambda i: (0, 0)),
            pl.BlockSpec((DIM, DIM), lambda i: (0, 0)),
        ],
        out_specs=pl.BlockSpec((_ROWS_BLK, DIM), lambda i: (i, 0)),
        out_shape=jax.ShapeDtypeStruct((N_NODES, DIM), jnp.float32),
    )(x, agg, nnb, nepn, wt, wb)


def kernel(node_feat, node_ids, edge_ids, num_nodes_per_edge,
           num_edges_per_node, num_neighbors, W0, W1):
    del num_nodes_per_edge
    nid = node_ids.astype(jnp.int32)
    eid = edge_ids.astype(jnp.int32)
    nnb = num_neighbors.reshape(N_NODES, 1)
    nepn = num_edges_per_node.reshape(N_NODES, 1)

    x = node_feat
    for W in (W0, W1):
        xt = x.reshape(N_NODES, NW, CPW).transpose(1, 2, 0).reshape(NW, N_NODES * CPW)
        aggt = _sc_aggregate(xt, nid, eid)
        agg = aggt.reshape(NW, CPW, N_NODES).transpose(2, 0, 1).reshape(N_NODES, DIM)
        x = _tc_layer(x, agg, nnb, nepn, W)
    return x
